# Initial kernel scaffold; baseline (speedup 1.0000x reference)
import jax
import jax.numpy as jnp
from jax.experimental import pallas as pl


def kernel(x, edge_index, W1, b1, W2, b2):
    raise NotImplementedError("write your pallas kernel here")


# speedup vs baseline: 79.0105x; 79.0105x over previous
import functools

import jax
import jax.numpy as jnp
from jax import lax
from jax.experimental import pallas as pl
from jax.experimental.pallas import tpu as pltpu
from jax.experimental.pallas import tpu_sc as plsc

N_NODES = 10000
N_EDGES = 320000
NT = 32
EPT = N_EDGES // NT
CHUNK = 2000
L = 16


def _wid():
    return lax.axis_index("s") * 2 + lax.axis_index("c")


def _zero_tables(refs):
    def body(i, carry):
        for r in refs:
            r[pl.ds(i * L, L)] = jnp.zeros((L,), jnp.float32)
        return carry
    lax.fori_loop(0, N_NODES // L, body, 0)


def _sc_degree(dst):
    mesh = plsc.VectorSubcoreMesh(core_axis_name="c", subcore_axis_name="s")

    @functools.partial(
        pl.kernel,
        mesh=mesh,
        out_type=jax.ShapeDtypeStruct((NT, N_NODES), jnp.float32),
        scratch_types=[
            pltpu.VMEM((CHUNK,), jnp.int32),
            pltpu.VMEM((N_NODES,), jnp.float32),
        ],
        compiler_params=pltpu.CompilerParams(needs_layout_passes=False),
    )
    def deg_kernel(dst_hbm, out_hbm, idx_v, tab_v):
        wid = _wid()
        _zero_tables([tab_v])
        ones = jnp.ones((L,), jnp.float32)
        base = pl.multiple_of(wid * EPT, 8)

        def chunk_body(ci, carry):
            off = pl.multiple_of(base + ci * CHUNK, 8)
            pltpu.sync_copy(dst_hbm.at[pl.ds(off, CHUNK)], idx_v)

            def ibody(i, c2):
                idx = idx_v[pl.ds(i * L, L)]
                plsc.addupdate_scatter(tab_v, [idx], ones)
                return c2
            lax.fori_loop(0, CHUNK // L, ibody, 0)
            return carry
        lax.fori_loop(0, EPT // CHUNK, chunk_body, 0)
        pltpu.sync_copy(tab_v, out_hbm.at[wid])

    return deg_kernel(dst)


def _sc_scatter(src, dst, g0, g1):
    mesh = plsc.VectorSubcoreMesh(core_axis_name="c", subcore_axis_name="s")

    @functools.partial(
        pl.kernel,
        mesh=mesh,
        out_type=[
            jax.ShapeDtypeStruct((NT, N_NODES), jnp.float32),
            jax.ShapeDtypeStruct((NT, N_NODES), jnp.float32),
        ],
        scratch_types=[
            pltpu.VMEM((CHUNK,), jnp.int32),
            pltpu.VMEM((CHUNK,), jnp.int32),
            pltpu.VMEM((N_NODES,), jnp.float32),
            pltpu.VMEM((N_NODES,), jnp.float32),
            pltpu.VMEM((N_NODES,), jnp.float32),
            pltpu.VMEM((N_NODES,), jnp.float32),
        ],
        compiler_params=pltpu.CompilerParams(needs_layout_passes=False),
    )
    def scat_kernel(src_hbm, dst_hbm, g0_hbm, g1_hbm, o0_hbm, o1_hbm,
                    s_v, d_v, t0, t1, a0, a1):
        wid = _wid()
        pltpu.sync_copy(g0_hbm, t0)
        pltpu.sync_copy(g1_hbm, t1)
        _zero_tables([a0, a1])
        base = pl.multiple_of(wid * EPT, 8)

        def chunk_body(ci, carry):
            off = pl.multiple_of(base + ci * CHUNK, 8)
            pltpu.sync_copy(src_hbm.at[pl.ds(off, CHUNK)], s_v)
            pltpu.sync_copy(dst_hbm.at[pl.ds(off, CHUNK)], d_v)

            def ibody(i, c2):
                s = s_v[pl.ds(i * L, L)]
                d = d_v[pl.ds(i * L, L)]
                v0 = plsc.load_gather(t0, [s])
                plsc.addupdate_scatter(a0, [d], v0)
                v1 = plsc.load_gather(t1, [s])
                plsc.addupdate_scatter(a1, [d], v1)
                return c2
            lax.fori_loop(0, CHUNK // L, ibody, 0)
            return carry
        lax.fori_loop(0, EPT // CHUNK, chunk_body, 0)
        pltpu.sync_copy(a0, o0_hbm.at[wid])
        pltpu.sync_copy(a1, o1_hbm.at[wid])

    return scat_kernel(src, dst, g0, g1)


def _tc_dinv(degp):
    def body(degp_ref, out_ref):
        s = jnp.sum(degp_ref[...], axis=0, keepdims=True)
        out_ref[...] = lax.rsqrt(s + 1.0)

    return pl.pallas_call(
        body,
        out_shape=jax.ShapeDtypeStruct((1, N_NODES), jnp.float32),
    )(degp)


def _tc_gs(x, W1, W2, dinv_col):
    def body(x_ref, w1_ref, w2_ref, dinv_ref, o0_ref, o1_ref):
        wc = jnp.dot(w1_ref[...], w2_ref[...],
                     preferred_element_type=jnp.float32)
        g = jnp.dot(x_ref[...], wc, preferred_element_type=jnp.float32)
        gs = g * dinv_ref[...]
        o0_ref[...] = gs[:, 0:1]
        o1_ref[...] = gs[:, 1:2]

    return pl.pallas_call(
        body,
        out_shape=[
            jax.ShapeDtypeStruct((N_NODES, 1), jnp.float32),
            jax.ShapeDtypeStruct((N_NODES, 1), jnp.float32),
        ],
    )(x, W1, W2, dinv_col)


def _tc_final(a0p, a1p, g0r, g1r, dinv_row, b1r, W2, b2r):
    def body(a0_ref, a1_ref, g0_ref, g1_ref, dinv_ref, b1_ref, w2_ref,
             b2_ref, o0_ref, o1_ref):
        bc = jnp.dot(b1_ref[...], w2_ref[...],
                     preferred_element_type=jnp.float32) + b2_ref[...]
        d = dinv_ref[...]
        s0 = jnp.sum(a0_ref[...], axis=0, keepdims=True) + g0_ref[...]
        s1 = jnp.sum(a1_ref[...], axis=0, keepdims=True) + g1_ref[...]
        o0_ref[...] = d * s0 + bc[:, 0:1]
        o1_ref[...] = d * s1 + bc[:, 1:2]

    return pl.pallas_call(
        body,
        out_shape=[
            jax.ShapeDtypeStruct((1, N_NODES), jnp.float32),
            jax.ShapeDtypeStruct((1, N_NODES), jnp.float32),
        ],
    )(a0p, a1p, g0r, g1r, dinv_row, b1r, W2, b2r)


def kernel(x, edge_index, W1, b1, W2, b2):
    src = edge_index[0].astype(jnp.int32)
    dst = edge_index[1].astype(jnp.int32)

    degp = _sc_degree(dst)
    dinv_row = _tc_dinv(degp)
    dinv_col = dinv_row.reshape(N_NODES, 1)

    g0c, g1c = _tc_gs(x, W1, W2, dinv_col)
    g0 = g0c.reshape(N_NODES)
    g1 = g1c.reshape(N_NODES)

    a0p, a1p = _sc_scatter(src, dst, g0, g1)

    o0, o1 = _tc_final(a0p, a1p,
                       g0.reshape(1, N_NODES), g1.reshape(1, N_NODES),
                       dinv_row, b1.reshape(1, -1), W2, b2.reshape(1, -1))
    return jnp.concatenate([o0.reshape(N_NODES, 1), o1.reshape(N_NODES, 1)],
                           axis=1)


# speedup vs baseline: 109.0996x; 1.3808x over previous
import functools

import jax
import jax.numpy as jnp
from jax import lax
from jax.experimental import pallas as pl
from jax.experimental.pallas import tpu as pltpu
from jax.experimental.pallas import tpu_sc as plsc

N_NODES = 10000
N_EDGES = 320000
NT = 32
EPT = N_EDGES // NT
L = 16


def _wid():
    return lax.axis_index("s") * 2 + lax.axis_index("c")


def _zero_tables(refs):
    def body(i, carry):
        for r in refs:
            r[pl.ds(i * L, L)] = jnp.zeros((L,), jnp.float32)
        return carry
    lax.fori_loop(0, N_NODES // L, body, 0)


def _sc_degree(dst):
    mesh = plsc.VectorSubcoreMesh(core_axis_name="c", subcore_axis_name="s")

    @functools.partial(
        pl.kernel,
        mesh=mesh,
        out_type=jax.ShapeDtypeStruct((NT, N_NODES), jnp.float32),
        scratch_types=[
            pltpu.VMEM((EPT,), jnp.int32),
            pltpu.VMEM((N_NODES,), jnp.float32),
        ],
        compiler_params=pltpu.CompilerParams(needs_layout_passes=False),
    )
    def deg_kernel(dst_hbm, out_hbm, idx_v, tab_v):
        wid = _wid()
        _zero_tables([tab_v])
        ones = jnp.ones((L,), jnp.float32)
        base = pl.multiple_of(wid * EPT, 8)
        pltpu.sync_copy(dst_hbm.at[pl.ds(base, EPT)], idx_v)

        def ibody(i, c2):
            idx = idx_v[pl.ds(i * L, L)]
            plsc.addupdate_scatter(tab_v, [idx], ones)
            return c2
        lax.fori_loop(0, EPT // L, ibody, 0)
        pltpu.sync_copy(tab_v, out_hbm.at[wid])

    return deg_kernel(dst)


def _sc_scatter(src, dst, g0, g1):
    mesh = plsc.VectorSubcoreMesh(core_axis_name="c", subcore_axis_name="s")

    @functools.partial(
        pl.kernel,
        mesh=mesh,
        out_type=[
            jax.ShapeDtypeStruct((NT, N_NODES), jnp.float32),
            jax.ShapeDtypeStruct((NT, N_NODES), jnp.float32),
        ],
        scratch_types=[
            pltpu.VMEM((EPT,), jnp.int32),
            pltpu.VMEM((EPT,), jnp.int32),
            pltpu.VMEM((N_NODES,), jnp.float32),
            pltpu.VMEM((N_NODES,), jnp.float32),
            pltpu.VMEM((N_NODES,), jnp.float32),
            pltpu.VMEM((N_NODES,), jnp.float32),
        ],
        compiler_params=pltpu.CompilerParams(needs_layout_passes=False),
    )
    def scat_kernel(src_hbm, dst_hbm, g0_hbm, g1_hbm, o0_hbm, o1_hbm,
                    s_v, d_v, t0, t1, a0, a1):
        wid = _wid()
        pltpu.sync_copy(g0_hbm, t0)
        pltpu.sync_copy(g1_hbm, t1)
        _zero_tables([a0, a1])
        base = pl.multiple_of(wid * EPT, 8)
        pltpu.sync_copy(src_hbm.at[pl.ds(base, EPT)], s_v)
        pltpu.sync_copy(dst_hbm.at[pl.ds(base, EPT)], d_v)

        def ibody(i, c2):
            s = s_v[pl.ds(i * L, L)]
            d = d_v[pl.ds(i * L, L)]
            v0 = plsc.load_gather(t0, [s])
            plsc.addupdate_scatter(a0, [d], v0)
            v1 = plsc.load_gather(t1, [s])
            plsc.addupdate_scatter(a1, [d], v1)
            return c2
        lax.fori_loop(0, EPT // L, ibody, 0)
        pltpu.sync_copy(a0, o0_hbm.at[wid])
        pltpu.sync_copy(a1, o1_hbm.at[wid])

    return scat_kernel(src, dst, g0, g1)


def _tc_matmul(x, W1, W2):
    def body(x_ref, w1_ref, w2_ref, o0_ref, o1_ref):
        wc = jnp.dot(w1_ref[...], w2_ref[...],
                     preferred_element_type=jnp.float32)
        gt = lax.dot_general(wc, x_ref[...], (((0,), (1,)), ((), ())),
                             preferred_element_type=jnp.float32)
        o0_ref[...] = gt[0:1, :]
        o1_ref[...] = gt[1:2, :]

    return pl.pallas_call(
        body,
        out_shape=[
            jax.ShapeDtypeStruct((1, N_NODES), jnp.float32),
            jax.ShapeDtypeStruct((1, N_NODES), jnp.float32),
        ],
    )(x, W1, W2)


def _tc_dinv_gs(degp, g0r, g1r):
    def body(degp_ref, g0_ref, g1_ref, dinv_ref, gs0_ref, gs1_ref):
        s = jnp.sum(degp_ref[...], axis=0, keepdims=True)
        d = lax.rsqrt(s + 1.0)
        dinv_ref[...] = d
        gs0_ref[...] = g0_ref[...] * d
        gs1_ref[...] = g1_ref[...] * d

    return pl.pallas_call(
        body,
        out_shape=[
            jax.ShapeDtypeStruct((1, N_NODES), jnp.float32),
            jax.ShapeDtypeStruct((1, N_NODES), jnp.float32),
            jax.ShapeDtypeStruct((1, N_NODES), jnp.float32),
        ],
    )(degp, g0r, g1r)


def _tc_final(a0p, a1p, gs0r, gs1r, dinv_row, b1r, W2, b2r):
    def body(a0_ref, a1_ref, gs0_ref, gs1_ref, dinv_ref, b1_ref, w2_ref,
             b2_ref, o0_ref, o1_ref):
        bc = jnp.dot(b1_ref[...], w2_ref[...],
                     preferred_element_type=jnp.float32) + b2_ref[...]
        d = dinv_ref[...]
        s0 = jnp.sum(a0_ref[...], axis=0, keepdims=True) + gs0_ref[...]
        s1 = jnp.sum(a1_ref[...], axis=0, keepdims=True) + gs1_ref[...]
        o0_ref[...] = d * s0 + bc[:, 0:1]
        o1_ref[...] = d * s1 + bc[:, 1:2]

    return pl.pallas_call(
        body,
        out_shape=[
            jax.ShapeDtypeStruct((1, N_NODES), jnp.float32),
            jax.ShapeDtypeStruct((1, N_NODES), jnp.float32),
        ],
    )(a0p, a1p, gs0r, gs1r, dinv_row, b1r, W2, b2r)


def kernel(x, edge_index, W1, b1, W2, b2):
    src = edge_index[0].astype(jnp.int32)
    dst = edge_index[1].astype(jnp.int32)

    g0r, g1r = _tc_matmul(x, W1, W2)
    degp = _sc_degree(dst)

    dinv_row, gs0r, gs1r = _tc_dinv_gs(degp, g0r, g1r)

    a0p, a1p = _sc_scatter(src, dst,
                           gs0r.reshape(N_NODES), gs1r.reshape(N_NODES))

    o0, o1 = _tc_final(a0p, a1p, gs0r, gs1r, dinv_row,
                       b1.reshape(1, -1), W2, b2.reshape(1, -1))
    return jnp.concatenate([o0.reshape(N_NODES, 1), o1.reshape(N_NODES, 1)],
                           axis=1)


# speedup vs baseline: 130.5154x; 1.1963x over previous
import functools

import jax
import jax.numpy as jnp
from jax import lax
from jax.experimental import pallas as pl
from jax.experimental.pallas import tpu as pltpu
from jax.experimental.pallas import tpu_sc as plsc

N_NODES = 10000
N_EDGES = 320000
NT = 32
EPT = N_EDGES // NT
L = 16


def _wid():
    return lax.axis_index("s") * 2 + lax.axis_index("c")


def _zero_tables(refs):
    @plsc.parallel_loop(0, N_NODES, step=L, unroll=8)
    def _(i):
        for r in refs:
            r[pl.ds(i, L)] = jnp.zeros((L,), jnp.float32)


def _sc_degree(dst):
    mesh = plsc.VectorSubcoreMesh(core_axis_name="c", subcore_axis_name="s")

    @functools.partial(
        pl.kernel,
        mesh=mesh,
        out_type=jax.ShapeDtypeStruct((NT, N_NODES), jnp.float32),
        scratch_types=[
            pltpu.VMEM((EPT,), jnp.int32),
            pltpu.VMEM((N_NODES,), jnp.float32),
        ],
        compiler_params=pltpu.CompilerParams(needs_layout_passes=False),
    )
    def deg_kernel(dst_hbm, out_hbm, idx_v, tab_v):
        wid = _wid()
        _zero_tables([tab_v])
        ones = jnp.ones((L,), jnp.float32)
        base = pl.multiple_of(wid * EPT, 8)
        pltpu.sync_copy(dst_hbm.at[pl.ds(base, EPT)], idx_v)

        @plsc.parallel_loop(0, EPT, step=L, unroll=8)
        def _(i):
            idx = idx_v[pl.ds(i, L)]
            plsc.addupdate_scatter(tab_v, [idx], ones)
        pltpu.sync_copy(tab_v, out_hbm.at[wid])

    return deg_kernel(dst)


def _sc_scatter(src, dst, g0, g1):
    mesh = plsc.VectorSubcoreMesh(core_axis_name="c", subcore_axis_name="s")

    @functools.partial(
        pl.kernel,
        mesh=mesh,
        out_type=[
            jax.ShapeDtypeStruct((NT, N_NODES), jnp.float32),
            jax.ShapeDtypeStruct((NT, N_NODES), jnp.float32),
        ],
        scratch_types=[
            pltpu.VMEM((EPT,), jnp.int32),
            pltpu.VMEM((EPT,), jnp.int32),
            pltpu.VMEM((N_NODES,), jnp.float32),
            pltpu.VMEM((N_NODES,), jnp.float32),
            pltpu.VMEM((N_NODES,), jnp.float32),
            pltpu.VMEM((N_NODES,), jnp.float32),
        ],
        compiler_params=pltpu.CompilerParams(needs_layout_passes=False),
    )
    def scat_kernel(src_hbm, dst_hbm, g0_hbm, g1_hbm, o0_hbm, o1_hbm,
                    s_v, d_v, t0, t1, a0, a1):
        wid = _wid()
        pltpu.sync_copy(g0_hbm, t0)
        pltpu.sync_copy(g1_hbm, t1)
        _zero_tables([a0, a1])
        base = pl.multiple_of(wid * EPT, 8)
        pltpu.sync_copy(src_hbm.at[pl.ds(base, EPT)], s_v)
        pltpu.sync_copy(dst_hbm.at[pl.ds(base, EPT)], d_v)

        @plsc.parallel_loop(0, EPT, step=L, unroll=8)
        def _(i):
            s = s_v[pl.ds(i, L)]
            d = d_v[pl.ds(i, L)]
            v0 = plsc.load_gather(t0, [s])
            plsc.addupdate_scatter(a0, [d], v0)
            v1 = plsc.load_gather(t1, [s])
            plsc.addupdate_scatter(a1, [d], v1)
        pltpu.sync_copy(a0, o0_hbm.at[wid])
        pltpu.sync_copy(a1, o1_hbm.at[wid])

    return scat_kernel(src, dst, g0, g1)


def _tc_matmul(x, W1, W2):
    def body(x_ref, w1_ref, w2_ref, o0_ref, o1_ref):
        wc = jnp.dot(w1_ref[...], w2_ref[...],
                     preferred_element_type=jnp.float32)
        gt = lax.dot_general(wc, x_ref[...], (((0,), (1,)), ((), ())),
                             preferred_element_type=jnp.float32)
        o0_ref[...] = gt[0:1, :]
        o1_ref[...] = gt[1:2, :]

    return pl.pallas_call(
        body,
        out_shape=[
            jax.ShapeDtypeStruct((1, N_NODES), jnp.float32),
            jax.ShapeDtypeStruct((1, N_NODES), jnp.float32),
        ],
    )(x, W1, W2)


def _tc_dinv_gs(degp, g0r, g1r):
    def body(degp_ref, g0_ref, g1_ref, dinv_ref, gs0_ref, gs1_ref):
        s = jnp.sum(degp_ref[...], axis=0, keepdims=True)
        d = lax.rsqrt(s + 1.0)
        dinv_ref[...] = d
        gs0_ref[...] = g0_ref[...] * d
        gs1_ref[...] = g1_ref[...] * d

    return pl.pallas_call(
        body,
        out_shape=[
            jax.ShapeDtypeStruct((1, N_NODES), jnp.float32),
            jax.ShapeDtypeStruct((1, N_NODES), jnp.float32),
            jax.ShapeDtypeStruct((1, N_NODES), jnp.float32),
        ],
    )(degp, g0r, g1r)


def _tc_final(a0p, a1p, gs0r, gs1r, dinv_row, b1r, W2, b2r):
    def body(a0_ref, a1_ref, gs0_ref, gs1_ref, dinv_ref, b1_ref, w2_ref,
             b2_ref, o0_ref, o1_ref):
        bc = jnp.dot(b1_ref[...], w2_ref[...],
                     preferred_element_type=jnp.float32) + b2_ref[...]
        d = dinv_ref[...]
        s0 = jnp.sum(a0_ref[...], axis=0, keepdims=True) + gs0_ref[...]
        s1 = jnp.sum(a1_ref[...], axis=0, keepdims=True) + gs1_ref[...]
        o0_ref[...] = d * s0 + bc[:, 0:1]
        o1_ref[...] = d * s1 + bc[:, 1:2]

    return pl.pallas_call(
        body,
        out_shape=[
            jax.ShapeDtypeStruct((1, N_NODES), jnp.float32),
            jax.ShapeDtypeStruct((1, N_NODES), jnp.float32),
        ],
    )(a0p, a1p, gs0r, gs1r, dinv_row, b1r, W2, b2r)


def kernel(x, edge_index, W1, b1, W2, b2):
    src = edge_index[0].astype(jnp.int32)
    dst = edge_index[1].astype(jnp.int32)

    g0r, g1r = _tc_matmul(x, W1, W2)
    degp = _sc_degree(dst)

    dinv_row, gs0r, gs1r = _tc_dinv_gs(degp, g0r, g1r)

    a0p, a1p = _sc_scatter(src, dst,
                           gs0r.reshape(N_NODES), gs1r.reshape(N_NODES))

    o0, o1 = _tc_final(a0p, a1p, gs0r, gs1r, dinv_row,
                       b1.reshape(1, -1), W2, b2.reshape(1, -1))
    return jnp.concatenate([o0.reshape(N_NODES, 1), o1.reshape(N_NODES, 1)],
                           axis=1)
